# baseline (speedup 1.0000x reference)
import functools

import jax
import jax.numpy as jnp
from jax import lax
from jax.experimental import pallas as pl
from jax.experimental.pallas import tpu as pltpu
from jax.experimental.pallas import tpu_sc as plsc

NC = 2
NS = 16
LANES = 16
CH = 128
SG = 8
SW = 128


def _t_kernel(x_ref, wa_ref, wb_ref, t_ref, *, hm):
    xb = x_ref[...]
    t_ref[:, :hm] = jnp.dot(xb, wa_ref[...], preferred_element_type=jnp.float32)
    t_ref[:, hm:] = jnp.dot(xb, wb_ref[...], preferred_element_type=jnp.float32)


def _r_kernel(ef_ref, we_ref, b_ref, r_ref, *, hm):
    rr = (jnp.dot(jnp.log1p(ef_ref[...]), we_ref[...],
                  preferred_element_type=jnp.float32) + b_ref[...])
    rr3 = rr.reshape(r_ref.shape[0], 2, rr.shape[1])
    r_ref[:, :rr.shape[1]] = rr3[:, 0, :]
    r_ref[:, rr.shape[1]:] = rr3[:, 1, :]


def _fin_kernel(sp_ref, x_ref, wm2_ref, bm2_ref, wu1a_ref, wu1b_ref,
                bu1_ref, wu2_ref, bu2_ref, o_ref, *, hm):
    s = sp_ref[0, :, :hm] + sp_ref[1, :, :hm]
    deg = sp_ref[0, :, hm:hm + 1] + sp_ref[1, :, hm:hm + 1]
    agg = (jnp.dot(s, wm2_ref[...], preferred_element_type=jnp.float32)
           + deg * bm2_ref[...])
    h2 = jnp.maximum(
        jnp.dot(x_ref[...], wu1a_ref[...], preferred_element_type=jnp.float32)
        + jnp.dot(agg, wu1b_ref[...], preferred_element_type=jnp.float32)
        + bu1_ref[...], 0.0)
    o_ref[...] = (jnp.dot(h2, wu2_ref[...], preferred_element_type=jnp.float32)
                  + bu2_ref[...])


def _edge_sc(src3, dst3, dsts3, t_tab, r2_tab, *, npad, hm, ep):
    nw = NC * NS
    ept = ep // nw
    k_chunks = ept // CH
    n_sg = k_chunks // SG
    rt = npad // NS
    nvr = hm // LANES
    tw = 2 * hm

    mesh = plsc.VectorSubcoreMesh(core_axis_name="c", subcore_axis_name="s",
                                  num_cores=NC, num_subcores=NS)

    @functools.partial(
        pl.kernel, mesh=mesh,
        out_type=jax.ShapeDtypeStruct((NC * npad, SW), jnp.float32),
        scratch_types=[
            pltpu.VMEM((SG, CH), jnp.int32),
            pltpu.VMEM((SG, CH), jnp.int32),
            pltpu.VMEM((SG, CH), jnp.int32),
            pltpu.VMEM((CH, tw), jnp.float32),
            pltpu.VMEM((CH, tw), jnp.float32),
            pltpu.VMEM((CH // 2, tw), jnp.float32),
            pltpu.VMEM_SHARED((npad, SW), jnp.float32),
            pltpu.SemaphoreType.DMA,
            pltpu.SemaphoreType.DMA,
            pltpu.SemaphoreType.DMA,
        ],
    )
    def edge_kernel(src_hbm, dst_hbm, dsts_hbm, t_hbm, r_hbm, out_hbm,
                    src_v, dst_v, dsts_v, p_v, q_v, r_v, s_acc,
                    sem_p, sem_q, sem_r):
        c = lax.axis_index("c")
        s = lax.axis_index("s")
        wid = c * NS + s

        zero = jnp.zeros((LANES,), jnp.float32)

        def zero_row(i, _):
            for j in range(tw // LANES):
                p_v[i, pl.ds(j * LANES, LANES)] = zero
            return 0

        lax.fori_loop(0, CH, zero_row, 0)

        for t in range(rt // CH):
            pltpu.sync_copy(p_v, s_acc.at[pl.ds(s * rt + t * CH, CH)])

        plsc.subcore_barrier()

        onehot = jnp.where(lax.iota(jnp.int32, LANES) == 0, 1.0, 0.0)

        def super_chunk(g, _):
            pltpu.sync_copy(src_hbm.at[wid, pl.ds(g * SG, SG)], src_v)
            pltpu.sync_copy(dst_hbm.at[wid, pl.ds(g * SG, SG)], dst_v)
            pltpu.sync_copy(dsts_hbm.at[wid, pl.ds(g * SG, SG)], dsts_v)

            def chunk(jj, _):
                rbase = (wid * k_chunks + g * SG + jj) * (CH // 2)
                cp = pltpu.async_copy(t_hbm.at[src_v.at[jj]], p_v, sem_p)
                cq = pltpu.async_copy(t_hbm.at[dst_v.at[jj]], q_v, sem_q)
                cr = pltpu.async_copy(r_hbm.at[pl.ds(rbase, CH // 2)], r_v,
                                      sem_r)
                cp.wait()
                cq.wait()
                cr.wait()

                def pair(k, _):
                    for half in range(2):
                        i = 2 * k + half
                        for v in range(nvr):
                            sl = pl.ds(v * LANES, LANES)
                            p_v[i, sl] = jnp.maximum(
                                p_v[i, sl]
                                + q_v[i, pl.ds(hm + v * LANES, LANES)]
                                + r_v[k, pl.ds(half * hm + v * LANES, LANES)],
                                0.0)
                        p_v[i, pl.ds(hm, LANES)] = onehot
                    return 0

                lax.fori_loop(0, CH // 2, pair, 0)
                pltpu.sync_copy(p_v, s_acc.at[dsts_v.at[jj]], add=True)
                return 0

            lax.fori_loop(0, SG, chunk, 0)
            return 0

        lax.fori_loop(0, n_sg, super_chunk, 0)

        plsc.subcore_barrier()

        pltpu.sync_copy(s_acc.at[pl.ds(s * rt, rt)],
                        out_hbm.at[pl.ds(c * npad + s * rt, rt)])

    return edge_kernel(src3, dst3, dsts3, t_tab, r2_tab)


def kernel(x, edge_features, Wm1, bm1, Wm2, bm2, Wu1, bu1, Wu2, bu2, edge_index):
    n, d = x.shape
    e, fe = edge_features.shape
    hm = Wm1.shape[1]
    dout = Wu2.shape[1]
    nw = NC * NS

    assert hm % LANES == 0 and 2 * hm == SW

    ep = -(-e // (nw * CH * SG)) * (nw * CH * SG)
    npad = -(-n // (NS * CH)) * (NS * CH)
    pad = ep - e
    e2 = ep // 2
    src_p = jnp.concatenate([edge_index[0], (jnp.arange(pad) * 37) % n])
    dst_p = jnp.concatenate([edge_index[1], (jnp.arange(pad) * 61) % n])
    dsts_p = jnp.concatenate(
        [edge_index[1], n + (jnp.arange(pad) % (npad - n))])

    bn = 2000
    assert n % bn == 0
    t_tab = pl.pallas_call(
        functools.partial(_t_kernel, hm=hm),
        grid=(n // bn,),
        in_specs=[
            pl.BlockSpec((bn, d), lambda i: (i, 0)),
            pl.BlockSpec((d, hm), lambda i: (0, 0)),
            pl.BlockSpec((d, hm), lambda i: (0, 0)),
        ],
        out_specs=pl.BlockSpec((bn, 2 * hm), lambda i: (i, 0)),
        out_shape=jax.ShapeDtypeStruct((n, 2 * hm), jnp.float32),
    )(x, Wm1[:d], Wm1[d:2 * d])

    be = 1280
    assert e2 % be == 0 and e % (2 * be) == 0
    nbe = e2 // be
    last_real = e // (2 * be) - 1
    r2_tab = pl.pallas_call(
        functools.partial(_r_kernel, hm=hm),
        grid=(nbe,),
        in_specs=[
            pl.BlockSpec((2 * be, fe),
                         lambda i: (jnp.minimum(i, last_real), 0)),
            pl.BlockSpec((fe, hm), lambda i: (0, 0)),
            pl.BlockSpec((1, hm), lambda i: (0, 0)),
        ],
        out_specs=pl.BlockSpec((be, 2 * hm), lambda i: (i, 0)),
        out_shape=jax.ShapeDtypeStruct((e2, 2 * hm), jnp.float32),
    )(edge_features, Wm1[2 * d:], bm1.reshape(1, hm))

    def _pack3(a):
        return a.reshape(nw, ep // (nw * CH), CH).astype(jnp.int32)

    sp = _edge_sc(_pack3(src_p), _pack3(dst_p), _pack3(dsts_p), t_tab, r2_tab,
                  npad=npad, hm=hm, ep=ep)
    sp = sp.reshape(NC, npad, SW)

    out = pl.pallas_call(
        functools.partial(_fin_kernel, hm=hm),
        grid=(n // bn,),
        in_specs=[
            pl.BlockSpec((NC, bn, SW), lambda i: (0, i, 0)),
            pl.BlockSpec((bn, d), lambda i: (i, 0)),
            pl.BlockSpec((hm, dout), lambda i: (0, 0)),
            pl.BlockSpec((1, dout), lambda i: (0, 0)),
            pl.BlockSpec((d, Wu1.shape[1]), lambda i: (0, 0)),
            pl.BlockSpec((d, Wu1.shape[1]), lambda i: (0, 0)),
            pl.BlockSpec((1, Wu1.shape[1]), lambda i: (0, 0)),
            pl.BlockSpec((Wu1.shape[1], dout), lambda i: (0, 0)),
            pl.BlockSpec((1, dout), lambda i: (0, 0)),
        ],
        out_specs=pl.BlockSpec((bn, dout), lambda i: (i, 0)),
        out_shape=jax.ShapeDtypeStruct((n, dout), jnp.float32),
    )(sp, x, Wm2, bm2.reshape(1, dout), Wu1[:d], Wu1[d:],
      bu1.reshape(1, -1), Wu2, bu2.reshape(1, dout))
    return out


# speedup vs baseline: 1.5330x; 1.5330x over previous
import functools

import jax
import jax.numpy as jnp
from jax import lax
from jax.experimental import pallas as pl
from jax.experimental.pallas import tpu as pltpu
from jax.experimental.pallas import tpu_sc as plsc

NC = 2
NS = 16
LANES = 16
CH = 128
SG = 8
SW = 128


def _t_kernel(x_ref, wa_ref, wb_ref, t_ref, *, hm):
    xb = x_ref[...]
    t_ref[:, :hm] = jnp.dot(xb, wa_ref[...], preferred_element_type=jnp.float32)
    t_ref[:, hm:] = jnp.dot(xb, wb_ref[...], preferred_element_type=jnp.float32)


def _r_kernel(eft_lo_ref, eft_hi_ref, we_ref, b_ref, r_ref, *, hm):
    dn = (((0,), (0,)), ((), ()))
    we = we_ref[...]
    b = b_ref[...]
    r_ref[:, :hm] = lax.dot_general(
        jnp.log1p(eft_lo_ref[...]), we, dn,
        preferred_element_type=jnp.float32) + b
    r_ref[:, hm:] = lax.dot_general(
        jnp.log1p(eft_hi_ref[...]), we, dn,
        preferred_element_type=jnp.float32) + b


def _fin_kernel(sp_ref, x_ref, wm2_ref, bm2_ref, wu1a_ref, wu1b_ref,
                bu1_ref, wu2_ref, bu2_ref, o_ref, *, hm):
    s = sp_ref[0, :, :hm] + sp_ref[1, :, :hm]
    deg = sp_ref[0, :, hm:hm + 1] + sp_ref[1, :, hm:hm + 1]
    agg = (jnp.dot(s, wm2_ref[...], preferred_element_type=jnp.float32)
           + deg * bm2_ref[...])
    h2 = jnp.maximum(
        jnp.dot(x_ref[...], wu1a_ref[...], preferred_element_type=jnp.float32)
        + jnp.dot(agg, wu1b_ref[...], preferred_element_type=jnp.float32)
        + bu1_ref[...], 0.0)
    o_ref[...] = (jnp.dot(h2, wu2_ref[...], preferred_element_type=jnp.float32)
                  + bu2_ref[...])


def _edge_sc(src3, dst3, dsts3, t_tab, r2_tab, *, npad, hm, ep):
    nw = NC * NS
    ept = ep // nw
    k_chunks = ept // CH
    n_sg = k_chunks // SG
    rt = npad // NS
    nvr = hm // LANES
    tw = 2 * hm

    mesh = plsc.VectorSubcoreMesh(core_axis_name="c", subcore_axis_name="s",
                                  num_cores=NC, num_subcores=NS)

    @functools.partial(
        pl.kernel, mesh=mesh,
        out_type=jax.ShapeDtypeStruct((NC * npad, SW), jnp.float32),
        scratch_types=[
            pltpu.VMEM((SG, CH), jnp.int32),
            pltpu.VMEM((SG, CH), jnp.int32),
            pltpu.VMEM((SG, CH), jnp.int32),
            pltpu.VMEM((CH, tw), jnp.float32),
            pltpu.VMEM((CH, tw), jnp.float32),
            pltpu.VMEM((CH // 2, tw), jnp.float32),
            pltpu.VMEM_SHARED((npad, SW), jnp.float32),
            pltpu.SemaphoreType.DMA,
            pltpu.SemaphoreType.DMA,
            pltpu.SemaphoreType.DMA,
        ],
    )
    def edge_kernel(src_hbm, dst_hbm, dsts_hbm, t_hbm, r_hbm, out_hbm,
                    src_v, dst_v, dsts_v, p_v, q_v, r_v, s_acc,
                    sem_p, sem_q, sem_r):
        c = lax.axis_index("c")
        s = lax.axis_index("s")
        wid = c * NS + s

        zero = jnp.zeros((LANES,), jnp.float32)

        def zero_row(i, _):
            for j in range(tw // LANES):
                p_v[i, pl.ds(j * LANES, LANES)] = zero
            return 0

        lax.fori_loop(0, CH, zero_row, 0)

        for t in range(rt // CH):
            pltpu.sync_copy(p_v, s_acc.at[pl.ds(s * rt + t * CH, CH)])

        plsc.subcore_barrier()

        onehot = jnp.where(lax.iota(jnp.int32, LANES) == 0, 1.0, 0.0)

        def super_chunk(g, _):
            pltpu.sync_copy(src_hbm.at[wid, pl.ds(g * SG, SG)], src_v)
            pltpu.sync_copy(dst_hbm.at[wid, pl.ds(g * SG, SG)], dst_v)
            pltpu.sync_copy(dsts_hbm.at[wid, pl.ds(g * SG, SG)], dsts_v)

            def chunk(jj, _):
                rbase = (wid * k_chunks + g * SG + jj) * (CH // 2)
                cp = pltpu.async_copy(t_hbm.at[src_v.at[jj]], p_v, sem_p)
                cq = pltpu.async_copy(t_hbm.at[dst_v.at[jj]], q_v, sem_q)
                cr = pltpu.async_copy(r_hbm.at[pl.ds(rbase, CH // 2)], r_v,
                                      sem_r)
                cp.wait()
                cq.wait()
                cr.wait()

                def pair(k, _):
                    for half in range(2):
                        i = half * (CH // 2) + k
                        for v in range(nvr):
                            sl = pl.ds(v * LANES, LANES)
                            p_v[i, sl] = jnp.maximum(
                                p_v[i, sl]
                                + q_v[i, pl.ds(hm + v * LANES, LANES)]
                                + r_v[k, pl.ds(half * hm + v * LANES, LANES)],
                                0.0)
                        p_v[i, pl.ds(hm, LANES)] = onehot
                    return 0

                lax.fori_loop(0, CH // 2, pair, 0)
                pltpu.sync_copy(p_v, s_acc.at[dsts_v.at[jj]], add=True)
                return 0

            lax.fori_loop(0, SG, chunk, 0)
            return 0

        lax.fori_loop(0, n_sg, super_chunk, 0)

        plsc.subcore_barrier()

        pltpu.sync_copy(s_acc.at[pl.ds(s * rt, rt)],
                        out_hbm.at[pl.ds(c * npad + s * rt, rt)])

    return edge_kernel(src3, dst3, dsts3, t_tab, r2_tab)


def kernel(x, edge_features, Wm1, bm1, Wm2, bm2, Wu1, bu1, Wu2, bu2, edge_index):
    n, d = x.shape
    e, fe = edge_features.shape
    hm = Wm1.shape[1]
    dout = Wu2.shape[1]
    nw = NC * NS

    assert hm % LANES == 0 and 2 * hm == SW

    ep = -(-e // (nw * CH * SG)) * (nw * CH * SG)
    npad = -(-n // (NS * CH)) * (NS * CH)
    pad = ep - e
    e2 = ep // 2
    src_p = jnp.concatenate([edge_index[0], (jnp.arange(pad) * 37) % n])
    dst_p = jnp.concatenate([edge_index[1], (jnp.arange(pad) * 61) % n])
    dsts_p = jnp.concatenate(
        [edge_index[1], n + (jnp.arange(pad) % (npad - n))])

    bn = 2000
    assert n % bn == 0
    t_tab = pl.pallas_call(
        functools.partial(_t_kernel, hm=hm),
        grid=(n // bn,),
        in_specs=[
            pl.BlockSpec((bn, d), lambda i: (i, 0)),
            pl.BlockSpec((d, hm), lambda i: (0, 0)),
            pl.BlockSpec((d, hm), lambda i: (0, 0)),
        ],
        out_specs=pl.BlockSpec((bn, 2 * hm), lambda i: (i, 0)),
        out_shape=jax.ShapeDtypeStruct((n, 2 * hm), jnp.float32),
    )(x, Wm1[:d], Wm1[d:2 * d])

    eft = edge_features.T
    be = 1280
    assert e2 % be == 0 and e % be == 0
    nbe = e2 // be
    hi0 = e2 // be
    last_real = e // be - 1
    r2_tab = pl.pallas_call(
        functools.partial(_r_kernel, hm=hm),
        grid=(nbe,),
        in_specs=[
            pl.BlockSpec((fe, be), lambda i: (0, i)),
            pl.BlockSpec((fe, be),
                         lambda i: (0, jnp.minimum(i + hi0, last_real))),
            pl.BlockSpec((fe, hm), lambda i: (0, 0)),
            pl.BlockSpec((1, hm), lambda i: (0, 0)),
        ],
        out_specs=pl.BlockSpec((be, 2 * hm), lambda i: (i, 0)),
        out_shape=jax.ShapeDtypeStruct((e2, 2 * hm), jnp.float32),
    )(eft, eft, Wm1[2 * d:], bm1.reshape(1, hm))

    h2c = CH // 2

    def _pack3(a):
        return jnp.concatenate(
            [a[:e2].reshape(-1, h2c), a[e2:].reshape(-1, h2c)],
            axis=1).reshape(nw, ep // (nw * CH), CH).astype(jnp.int32)

    sp = _edge_sc(_pack3(src_p), _pack3(dst_p), _pack3(dsts_p), t_tab, r2_tab,
                  npad=npad, hm=hm, ep=ep)
    sp = sp.reshape(NC, npad, SW)

    out = pl.pallas_call(
        functools.partial(_fin_kernel, hm=hm),
        grid=(n // bn,),
        in_specs=[
            pl.BlockSpec((NC, bn, SW), lambda i: (0, i, 0)),
            pl.BlockSpec((bn, d), lambda i: (i, 0)),
            pl.BlockSpec((hm, dout), lambda i: (0, 0)),
            pl.BlockSpec((1, dout), lambda i: (0, 0)),
            pl.BlockSpec((d, Wu1.shape[1]), lambda i: (0, 0)),
            pl.BlockSpec((d, Wu1.shape[1]), lambda i: (0, 0)),
            pl.BlockSpec((1, Wu1.shape[1]), lambda i: (0, 0)),
            pl.BlockSpec((Wu1.shape[1], dout), lambda i: (0, 0)),
            pl.BlockSpec((1, dout), lambda i: (0, 0)),
        ],
        out_specs=pl.BlockSpec((bn, dout), lambda i: (i, 0)),
        out_shape=jax.ShapeDtypeStruct((n, dout), jnp.float32),
    )(sp, x, Wm2, bm2.reshape(1, dout), Wu1[:d], Wu1[d:],
      bu1.reshape(1, -1), Wu2, bu2.reshape(1, dout))
    return out
